# baseline (speedup 1.0000x reference)
import functools

import numpy as np
import jax
import jax.numpy as jnp
from jax.experimental import pallas as pl
from jax.experimental.pallas import tpu as pltpu


def _make_paths(i):
    intpath = []
    for m in range(2 ** i):
        for n in range(2 ** i):
            if m == 0 and n == 0:
                continue
            mm = int(-(m + 1) / 2) if m % 2 == 1 else int(m / 2)
            nn = int(-(n + 1) / 2) if n % 2 == 1 else int(n / 2)
            logx1 = logy1 = ch1 = depthx1 = depthy1 = sign1 = None
            logx2 = logy2 = ch2 = depthx2 = depthy2 = sign2 = None
            if nn < 0:
                logx1, logy1, ch1, depthx1, depthy1, sign1 = nn, mm, 0, nn + 1, mm, -1
            elif nn > 0:
                logx1, logy1, ch1, depthx1, depthy1, sign1 = nn - 1, mm, 0, nn - 1, mm, 1
            if mm > 0:
                logx2, logy2, ch2, depthx2, depthy2, sign2 = nn, mm - 1, 1, nn, mm - 1, 1
            elif mm < 0:
                logx2, logy2, ch2, depthx2, depthy2, sign2 = nn, mm, 1, nn, mm + 1, -1
            if nn == 0:
                logx1, logy1, ch1, depthx1, depthy1, sign1 = logx2, logy2, ch2, depthx2, depthy2, sign2
            if mm == 0:
                logx2, logy2, ch2, depthx2, depthy2, sign2 = logx1, logy1, ch1, depthx1, depthy1, sign1
            intpath.append((logx1, logy1, ch1, depthx1, depthy1, sign1,
                            logx2, logy2, ch2, depthx2, depthy2, sign2, mm, nn))
    return intpath


@functools.lru_cache(maxsize=None)
def _coeff_matrix(scale_s):
    bs = 2 ** scale_s
    c = bs // 2
    nslot = 2 * bs * bs
    C = np.zeros((bs, bs, nslot), dtype=np.float64)
    for p in _make_paths(scale_s):
        (lx1, ly1, ch1, dx1, dy1, s1, lx2, ly2, ch2, dx2, dy2, s2, mm, nn) = p
        v = (C[c + dy1, c + dx1] + C[c + dy2, c + dx2]) / 2.0
        v = v.copy()
        v[ch1 * bs * bs + (c + ly1) * bs + (c + lx1)] += s1 / 2.0
        v[ch2 * bs * bs + (c + ly2) * bs + (c + lx2)] += s2 / 2.0
        C[c + mm, c + nn] = v
    return C.reshape(bs, bs, 2, bs, bs)


@functools.lru_cache(maxsize=None)
def _shift_groups(scale_s):
    bs = 2 ** scale_s
    A = _coeff_matrix(scale_s)
    groups = {}
    for py in range(bs):
        for px in range(bs):
            for ch in range(2):
                for dy in range(bs):
                    for dx in range(bs):
                        v = A[py, px, ch, dy, dx]
                        if v != 0.0:
                            key = (ch, py - dy, px - dx)
                            groups.setdefault(key, np.zeros((bs, bs), np.float32))[py, px] = v
    return tuple(sorted(groups.items()))


def _sincos(x):
    two_over_pi = 0.6366197723675814
    dp1 = 1.5703125
    dp2 = 4.837512969970703125e-4
    dp3 = 7.549789948768648e-8
    nf = jnp.floor(x * two_over_pi + 0.5)
    k = nf.astype(jnp.int32)
    r = ((x - nf * dp1) - nf * dp2) - nf * dp3
    r2 = r * r
    s = r + r * r2 * (-1.6666654611e-1
                      + r2 * (8.3321608736e-3 + r2 * (-1.9515295891e-4)))
    c = 1.0 - 0.5 * r2 + r2 * r2 * (4.166664568298827e-2
                                    + r2 * (-1.388731625493765e-3
                                            + r2 * 2.443315711809948e-5))
    swap = (k & 1) != 0
    ssel = jnp.where(swap, c, s)
    csel = jnp.where(swap, s, c)
    sinx = jnp.where((k & 2) != 0, -ssel, ssel)
    cosx = jnp.where(((k + 1) & 2) != 0, -csel, csel)
    return sinx, cosx


def _fused_kernel(av_ref, dl_ref, pp_ref, pat_ref, wide_ref, sel_ref, out_ref, *,
                  rows, bs, group_meta):
    t = pl.program_id(1)
    yf = (jax.lax.broadcasted_iota(jnp.int32, (rows, 512), 0)
          + t * rows).astype(jnp.float32)
    xf = jax.lax.broadcasted_iota(jnp.int32, (rows, 512), 1).astype(jnp.float32)

    fx = pp_ref[0, 0, 0]
    bx = pp_ref[0, 0, 1]
    fy = pp_ref[0, 0, 2]
    by = pp_ref[0, 0, 3]
    shift = pp_ref[0, 0, 4]

    def loggrad(av, main, fm, bm, perp, fp, bp):
        pn = (perp - bp) / fp
        p1 = pn * pn + 1.0
        q1 = -(main - bm) / fm
        sn, cs = _sincos(av)
        t1 = sn * q1 + p1 * cs
        t2 = t1 - sn * (1.0 / fm)
        n1 = jnp.clip(jnp.abs(t1), 1e-6, None)
        n2 = jnp.clip(jnp.abs(t2), 1e-6, None)
        return jnp.clip(jnp.log(n1 / n2), -10.0, 10.0)

    lg0 = loggrad(av_ref[0, 0], xf, fx, bx, yf, fy, by)
    lg1 = loggrad(av_ref[0, 1], yf, fy, by, xf, fx, bx)
    planes = (lg0, lg1)

    dll = dl_ref[0, 0] * jnp.exp(shift)
    drow = jnp.dot(sel_ref[...], dll, preferred_element_type=jnp.float32,
                   precision=jax.lax.Precision.HIGHEST)
    dbc = jnp.dot(drow, wide_ref[...], preferred_element_type=jnp.float32,
                  precision=jax.lax.Precision.HIGHEST)

    acc = None
    gidx = 0
    last = (None, None, None)
    rolled = None
    for (ch, sy, sx) in group_meta:
        if (ch, sy) != last[:2]:
            rolled = jnp.roll(planes[ch], sy, axis=0) if sy else planes[ch]
            last = (ch, sy, None)
        term = jnp.roll(rolled, sx, axis=1) if sx else rolled
        contrib = pat_ref[gidx] * term
        acc = contrib if acc is None else acc + contrib
        gidx += 1

    out_ref[0, 0] = jnp.exp(acc) * dbc


def kernel(depthmaplow, ang, intrinsic, scale):
    B, _, H, W = ang.shape
    cuh, cuw = depthmaplow.shape[2], depthmaplow.shape[3]
    scale_s = int(round(np.log2(H // cuh)))
    bs = 2 ** scale_s
    rows = 384

    groups = _shift_groups(scale_s)
    group_meta = tuple(k for k, _ in groups)
    pats = np.stack([np.tile(p, (rows // bs, W // bs)) for _, p in groups])
    pats = jnp.asarray(pats)

    wide = (np.arange(W)[None, :] // bs == np.arange(W // bs)[:, None])
    wide = jnp.asarray(wide.astype(np.float32))

    selm = (np.arange(rows)[:, None] // bs == np.arange(rows // bs)[None, :])
    selm = jnp.asarray(selm.astype(np.float32))

    fxv = intrinsic[:, 0, 0]
    bxv = intrinsic[:, 0, 2]
    fyv = intrinsic[:, 1, 1]
    byv = intrinsic[:, 1, 2]
    shiftv = jnp.full((B,), jnp.asarray(scale, jnp.float32) - scale_s)
    zeros = jnp.zeros((B,), jnp.float32)
    params = jnp.stack([fxv, bxv, fyv, byv, shiftv, zeros, zeros, zeros],
                       axis=1).reshape(B, 1, 8)

    ngroups = len(groups)
    out = pl.pallas_call(
        functools.partial(_fused_kernel, rows=rows, bs=bs, group_meta=group_meta),
        grid=(B, H // rows),
        in_specs=[
            pl.BlockSpec((1, 2, rows, W), lambda b, t: (b, 0, t, 0)),
            pl.BlockSpec((1, 1, rows // bs, W // bs), lambda b, t: (b, 0, t, 0)),
            pl.BlockSpec((1, 1, 8), lambda b, t: (b, 0, 0)),
            pl.BlockSpec((ngroups, rows, W), lambda b, t: (0, 0, 0)),
            pl.BlockSpec((W // bs, W), lambda b, t: (0, 0)),
            pl.BlockSpec((rows, rows // bs), lambda b, t: (0, 0)),
        ],
        out_specs=pl.BlockSpec((1, 1, rows, W), lambda b, t: (b, 0, t, 0)),
        out_shape=jax.ShapeDtypeStruct((B, 1, H, W), jnp.float32),
        compiler_params=pltpu.CompilerParams(
            dimension_semantics=("parallel", "parallel")),
    )(ang, depthmaplow, params, pats, wide, selm)
    return out


# speedup vs baseline: 1.1491x; 1.1491x over previous
import functools

import numpy as np
import jax
import jax.numpy as jnp
from jax.experimental import pallas as pl
from jax.experimental.pallas import tpu as pltpu


def _make_paths(i):
    intpath = []
    for m in range(2 ** i):
        for n in range(2 ** i):
            if m == 0 and n == 0:
                continue
            mm = int(-(m + 1) / 2) if m % 2 == 1 else int(m / 2)
            nn = int(-(n + 1) / 2) if n % 2 == 1 else int(n / 2)
            logx1 = logy1 = ch1 = depthx1 = depthy1 = sign1 = None
            logx2 = logy2 = ch2 = depthx2 = depthy2 = sign2 = None
            if nn < 0:
                logx1, logy1, ch1, depthx1, depthy1, sign1 = nn, mm, 0, nn + 1, mm, -1
            elif nn > 0:
                logx1, logy1, ch1, depthx1, depthy1, sign1 = nn - 1, mm, 0, nn - 1, mm, 1
            if mm > 0:
                logx2, logy2, ch2, depthx2, depthy2, sign2 = nn, mm - 1, 1, nn, mm - 1, 1
            elif mm < 0:
                logx2, logy2, ch2, depthx2, depthy2, sign2 = nn, mm, 1, nn, mm + 1, -1
            if nn == 0:
                logx1, logy1, ch1, depthx1, depthy1, sign1 = logx2, logy2, ch2, depthx2, depthy2, sign2
            if mm == 0:
                logx2, logy2, ch2, depthx2, depthy2, sign2 = logx1, logy1, ch1, depthx1, depthy1, sign1
            intpath.append((logx1, logy1, ch1, depthx1, depthy1, sign1,
                            logx2, logy2, ch2, depthx2, depthy2, sign2, mm, nn))
    return intpath


@functools.lru_cache(maxsize=None)
def _coeff_matrix(scale_s):
    bs = 2 ** scale_s
    c = bs // 2
    nslot = 2 * bs * bs
    C = np.zeros((bs, bs, nslot), dtype=np.float64)
    for p in _make_paths(scale_s):
        (lx1, ly1, ch1, dx1, dy1, s1, lx2, ly2, ch2, dx2, dy2, s2, mm, nn) = p
        v = (C[c + dy1, c + dx1] + C[c + dy2, c + dx2]) / 2.0
        v = v.copy()
        v[ch1 * bs * bs + (c + ly1) * bs + (c + lx1)] += s1 / 2.0
        v[ch2 * bs * bs + (c + ly2) * bs + (c + lx2)] += s2 / 2.0
        C[c + mm, c + nn] = v
    return C.reshape(bs, bs, 2, bs, bs)


@functools.lru_cache(maxsize=None)
def _shift_groups(scale_s):
    bs = 2 ** scale_s
    A = _coeff_matrix(scale_s)
    groups = {}
    for py in range(bs):
        for px in range(bs):
            for ch in range(2):
                for dy in range(bs):
                    for dx in range(bs):
                        v = A[py, px, ch, dy, dx]
                        if v != 0.0:
                            key = (ch, py - dy, px - dx)
                            groups.setdefault(key, np.zeros((bs, bs), np.float32))[py, px] = v
    return tuple(sorted(groups.items()))


def _sincos(x):
    two_over_pi = 0.6366197723675814
    dp1 = 1.5703125
    dp2 = 4.837512969970703125e-4
    dp3 = 7.549789948768648e-8
    nf = jnp.floor(x * two_over_pi + 0.5)
    k = nf.astype(jnp.int32)
    r = ((x - nf * dp1) - nf * dp2) - nf * dp3
    r2 = r * r
    s = r + r * r2 * (-1.6666654611e-1
                      + r2 * (8.3321608736e-3 + r2 * (-1.9515295891e-4)))
    c = 1.0 - 0.5 * r2 + r2 * r2 * (4.166664568298827e-2
                                    + r2 * (-1.388731625493765e-3
                                            + r2 * 2.443315711809948e-5))
    swap = (k & 1) != 0
    ssel = jnp.where(swap, c, s)
    csel = jnp.where(swap, s, c)
    sinx = jnp.where((k & 2) != 0, -ssel, ssel)
    cosx = jnp.where(((k + 1) & 2) != 0, -csel, csel)
    return sinx, cosx


def _fused_kernel(av_ref, dl_ref, pp_ref, pat_ref, wide_ref, sel_ref, out_ref, *,
                  rows, bs, group_meta):
    t = pl.program_id(1)
    yf = (jax.lax.broadcasted_iota(jnp.int32, (rows, 512), 0)
          + t * rows).astype(jnp.float32)
    xf = jax.lax.broadcasted_iota(jnp.int32, (rows, 512), 1).astype(jnp.float32)

    fx = pp_ref[0, 0, 0]
    bx = pp_ref[0, 0, 1]
    fy = pp_ref[0, 0, 2]
    by = pp_ref[0, 0, 3]
    shift = pp_ref[0, 0, 4]

    def loggrad(av, main, fm, bm, perp, fp, bp):
        pn = (perp - bp) / fp
        p1 = pn * pn + 1.0
        q1 = -(main - bm) / fm
        sn, cs = _sincos(av)
        t1 = sn * q1 + p1 * cs
        t2 = t1 - sn * (1.0 / fm)
        n1 = jnp.clip(jnp.abs(t1), 1e-6, None)
        n2 = jnp.clip(jnp.abs(t2), 1e-6, None)
        return jnp.clip(jnp.log(n1 / n2), -10.0, 10.0)

    lg0 = loggrad(av_ref[0, 0], xf, fx, bx, yf, fy, by)
    lg1 = loggrad(av_ref[0, 1], yf, fy, by, xf, fx, bx)
    planes = (lg0, lg1)

    dll = dl_ref[0, 0] * jnp.exp(shift)
    drow = jnp.dot(sel_ref[...], dll, preferred_element_type=jnp.float32,
                   precision=jax.lax.Precision.HIGHEST)
    dbc = jnp.dot(drow, wide_ref[...], preferred_element_type=jnp.float32,
                  precision=jax.lax.Precision.HIGHEST)

    acc = None
    gidx = 0
    last = (None, None, None)
    rolled = None
    for (ch, sy, sx) in group_meta:
        if (ch, sy) != last[:2]:
            rolled = jnp.roll(planes[ch], sy, axis=0) if sy else planes[ch]
            last = (ch, sy, None)
        term = jnp.roll(rolled, sx, axis=1) if sx else rolled
        contrib = pat_ref[gidx] * term
        acc = contrib if acc is None else acc + contrib
        gidx += 1

    out_ref[0, 0] = jnp.exp(acc) * dbc


def kernel(depthmaplow, ang, intrinsic, scale):
    B, _, H, W = ang.shape
    cuh, cuw = depthmaplow.shape[2], depthmaplow.shape[3]
    scale_s = int(round(np.log2(H // cuh)))
    bs = 2 ** scale_s
    rows = 128

    groups = _shift_groups(scale_s)
    group_meta = tuple(k for k, _ in groups)
    pats = np.stack([np.tile(p, (rows // bs, W // bs)) for _, p in groups])
    pats = jnp.asarray(pats)

    wide = (np.arange(W)[None, :] // bs == np.arange(W // bs)[:, None])
    wide = jnp.asarray(wide.astype(np.float32))

    selm = (np.arange(rows)[:, None] // bs == np.arange(rows // bs)[None, :])
    selm = jnp.asarray(selm.astype(np.float32))

    fxv = intrinsic[:, 0, 0]
    bxv = intrinsic[:, 0, 2]
    fyv = intrinsic[:, 1, 1]
    byv = intrinsic[:, 1, 2]
    shiftv = jnp.full((B,), jnp.asarray(scale, jnp.float32) - scale_s)
    zeros = jnp.zeros((B,), jnp.float32)
    params = jnp.stack([fxv, bxv, fyv, byv, shiftv, zeros, zeros, zeros],
                       axis=1).reshape(B, 1, 8)

    ngroups = len(groups)
    out = pl.pallas_call(
        functools.partial(_fused_kernel, rows=rows, bs=bs, group_meta=group_meta),
        grid=(B, H // rows),
        in_specs=[
            pl.BlockSpec((1, 2, rows, W), lambda b, t: (b, 0, t, 0)),
            pl.BlockSpec((1, 1, rows // bs, W // bs), lambda b, t: (b, 0, t, 0)),
            pl.BlockSpec((1, 1, 8), lambda b, t: (b, 0, 0)),
            pl.BlockSpec((ngroups, rows, W), lambda b, t: (0, 0, 0)),
            pl.BlockSpec((W // bs, W), lambda b, t: (0, 0)),
            pl.BlockSpec((rows, rows // bs), lambda b, t: (0, 0)),
        ],
        out_specs=pl.BlockSpec((1, 1, rows, W), lambda b, t: (b, 0, t, 0)),
        out_shape=jax.ShapeDtypeStruct((B, 1, H, W), jnp.float32),
        compiler_params=pltpu.CompilerParams(
            dimension_semantics=("parallel", "parallel")),
    )(ang, depthmaplow, params, pats, wide, selm)
    return out


# speedup vs baseline: 1.2445x; 1.0830x over previous
import functools

import numpy as np
import jax
import jax.numpy as jnp
from jax.experimental import pallas as pl
from jax.experimental.pallas import tpu as pltpu


def _make_paths(i):
    intpath = []
    for m in range(2 ** i):
        for n in range(2 ** i):
            if m == 0 and n == 0:
                continue
            mm = int(-(m + 1) / 2) if m % 2 == 1 else int(m / 2)
            nn = int(-(n + 1) / 2) if n % 2 == 1 else int(n / 2)
            logx1 = logy1 = ch1 = depthx1 = depthy1 = sign1 = None
            logx2 = logy2 = ch2 = depthx2 = depthy2 = sign2 = None
            if nn < 0:
                logx1, logy1, ch1, depthx1, depthy1, sign1 = nn, mm, 0, nn + 1, mm, -1
            elif nn > 0:
                logx1, logy1, ch1, depthx1, depthy1, sign1 = nn - 1, mm, 0, nn - 1, mm, 1
            if mm > 0:
                logx2, logy2, ch2, depthx2, depthy2, sign2 = nn, mm - 1, 1, nn, mm - 1, 1
            elif mm < 0:
                logx2, logy2, ch2, depthx2, depthy2, sign2 = nn, mm, 1, nn, mm + 1, -1
            if nn == 0:
                logx1, logy1, ch1, depthx1, depthy1, sign1 = logx2, logy2, ch2, depthx2, depthy2, sign2
            if mm == 0:
                logx2, logy2, ch2, depthx2, depthy2, sign2 = logx1, logy1, ch1, depthx1, depthy1, sign1
            intpath.append((logx1, logy1, ch1, depthx1, depthy1, sign1,
                            logx2, logy2, ch2, depthx2, depthy2, sign2, mm, nn))
    return intpath


@functools.lru_cache(maxsize=None)
def _coeff_matrix(scale_s):
    bs = 2 ** scale_s
    c = bs // 2
    nslot = 2 * bs * bs
    C = np.zeros((bs, bs, nslot), dtype=np.float64)
    for p in _make_paths(scale_s):
        (lx1, ly1, ch1, dx1, dy1, s1, lx2, ly2, ch2, dx2, dy2, s2, mm, nn) = p
        v = (C[c + dy1, c + dx1] + C[c + dy2, c + dx2]) / 2.0
        v = v.copy()
        v[ch1 * bs * bs + (c + ly1) * bs + (c + lx1)] += s1 / 2.0
        v[ch2 * bs * bs + (c + ly2) * bs + (c + lx2)] += s2 / 2.0
        C[c + mm, c + nn] = v
    return C.reshape(bs, bs, 2, bs, bs)


@functools.lru_cache(maxsize=None)
def _shift_groups(scale_s):
    bs = 2 ** scale_s
    A = _coeff_matrix(scale_s)
    groups = {}
    for py in range(bs):
        for px in range(bs):
            for ch in range(2):
                for dy in range(bs):
                    for dx in range(bs):
                        v = A[py, px, ch, dy, dx]
                        if v != 0.0:
                            key = (ch, py - dy, px - dx)
                            groups.setdefault(key, np.zeros((bs, bs), np.float32))[py, px] = v
    return tuple(sorted(groups.items()))


def _sincos(x):
    two_over_pi = 0.6366197723675814
    dp1 = 1.5703125
    dp2 = 4.837512969970703125e-4
    dp3 = 7.549789948768648e-8
    nf = jnp.floor(x * two_over_pi + 0.5)
    k = nf.astype(jnp.int32)
    r = ((x - nf * dp1) - nf * dp2) - nf * dp3
    r2 = r * r
    s = r + r * r2 * (-1.6666654611e-1
                      + r2 * (8.3321608736e-3 + r2 * (-1.9515295891e-4)))
    c = 1.0 - 0.5 * r2 + r2 * r2 * (4.166664568298827e-2
                                    + r2 * (-1.388731625493765e-3
                                            + r2 * 2.443315711809948e-5))
    swap = (k & 1) != 0
    ssel = jnp.where(swap, c, s)
    csel = jnp.where(swap, s, c)
    sinx = jnp.where((k & 2) != 0, -ssel, ssel)
    cosx = jnp.where(((k + 1) & 2) != 0, -csel, csel)
    return sinx, cosx


def _fused_kernel(av_ref, dl_ref, pp_ref, pat_ref, wide_ref, sel_ref, out_ref, *,
                  rows, bs, group_meta):
    t = pl.program_id(1)
    yf = (jax.lax.broadcasted_iota(jnp.int32, (rows, 512), 0)
          + t * rows).astype(jnp.float32)
    xf = jax.lax.broadcasted_iota(jnp.int32, (rows, 512), 1).astype(jnp.float32)

    fx = pp_ref[0, 0, 0]
    bx = pp_ref[0, 0, 1]
    fy = pp_ref[0, 0, 2]
    by = pp_ref[0, 0, 3]
    shift = pp_ref[0, 0, 4]

    def loggrad(av, main, fm, bm, perp, fp, bp):
        pn = (perp - bp) / fp
        p1 = pn * pn + 1.0
        q1 = -(main - bm) / fm
        sn, cs = _sincos(av)
        t1 = sn * q1 + p1 * cs
        t2 = t1 - sn * (1.0 / fm)
        n1 = jnp.clip(jnp.abs(t1), 1e-6, None)
        n2 = jnp.clip(jnp.abs(t2), 1e-6, None)
        return jnp.clip(jnp.log(n1 / n2), -10.0, 10.0)

    lg0 = loggrad(av_ref[0, 0], xf, fx, bx, yf, fy, by)
    lg1 = loggrad(av_ref[0, 1], yf, fy, by, xf, fx, bx)
    planes = (lg0, lg1)

    dll = dl_ref[0, 0] * jnp.exp(shift)
    drow = jnp.dot(sel_ref[...], dll, preferred_element_type=jnp.float32,
                   precision=jax.lax.Precision.HIGHEST)
    dbc = jnp.dot(drow, wide_ref[...], preferred_element_type=jnp.float32,
                  precision=jax.lax.Precision.HIGHEST)

    acc = None
    gidx = 0
    last = (None, None, None)
    rolled = None
    for (ch, sy, sx) in group_meta:
        if (ch, sy) != last[:2]:
            rolled = jnp.roll(planes[ch], sy, axis=0) if sy else planes[ch]
            last = (ch, sy, None)
        term = jnp.roll(rolled, sx, axis=1) if sx else rolled
        pat = jnp.broadcast_to(pat_ref[gidx][None], (rows // 8, 8, 512))
        pat = pat.reshape(rows, 512)
        contrib = pat * term
        acc = contrib if acc is None else acc + contrib
        gidx += 1

    out_ref[0, 0] = jnp.exp(acc) * dbc


def kernel(depthmaplow, ang, intrinsic, scale):
    B, _, H, W = ang.shape
    cuh, cuw = depthmaplow.shape[2], depthmaplow.shape[3]
    scale_s = int(round(np.log2(H // cuh)))
    bs = 2 ** scale_s
    rows = 128

    groups = _shift_groups(scale_s)
    group_meta = tuple(k for k, _ in groups)
    pats = np.stack([np.tile(p, (8 // bs, W // bs)) for _, p in groups])
    pats = jnp.asarray(pats)

    wide = (np.arange(W)[None, :] // bs == np.arange(W // bs)[:, None])
    wide = jnp.asarray(wide.astype(np.float32))

    selm = (np.arange(rows)[:, None] // bs == np.arange(rows // bs)[None, :])
    selm = jnp.asarray(selm.astype(np.float32))

    fxv = intrinsic[:, 0, 0]
    bxv = intrinsic[:, 0, 2]
    fyv = intrinsic[:, 1, 1]
    byv = intrinsic[:, 1, 2]
    shiftv = jnp.full((B,), jnp.asarray(scale, jnp.float32) - scale_s)
    zeros = jnp.zeros((B,), jnp.float32)
    params = jnp.stack([fxv, bxv, fyv, byv, shiftv, zeros, zeros, zeros],
                       axis=1).reshape(B, 1, 8)

    ngroups = len(groups)
    out = pl.pallas_call(
        functools.partial(_fused_kernel, rows=rows, bs=bs, group_meta=group_meta),
        grid=(B, H // rows),
        in_specs=[
            pl.BlockSpec((1, 2, rows, W), lambda b, t: (b, 0, t, 0)),
            pl.BlockSpec((1, 1, rows // bs, W // bs), lambda b, t: (b, 0, t, 0)),
            pl.BlockSpec((1, 1, 8), lambda b, t: (b, 0, 0)),
            pl.BlockSpec((ngroups, 8, W), lambda b, t: (0, 0, 0)),
            pl.BlockSpec((W // bs, W), lambda b, t: (0, 0)),
            pl.BlockSpec((rows, rows // bs), lambda b, t: (0, 0)),
        ],
        out_specs=pl.BlockSpec((1, 1, rows, W), lambda b, t: (b, 0, t, 0)),
        out_shape=jax.ShapeDtypeStruct((B, 1, H, W), jnp.float32),
        compiler_params=pltpu.CompilerParams(
            dimension_semantics=("parallel", "parallel")),
    )(ang, depthmaplow, params, pats, wide, selm)
    return out
